# baseline (speedup 1.0000x reference)
import functools

import jax
import jax.numpy as jnp
from jax import lax
from jax.experimental import pallas as pl
from jax.experimental.pallas import tpu as pltpu
from jax.experimental.pallas import tpu_sc as plsc

_K = 20
_NB = 4
_P = 1024
_N = _NB * _P


def _knn_proj_body(x_ref, wc_ref, wb_ref, b1_ref, c_ref, n_ref, idx_ref):
    xb = x_ref[...]
    c_ref[...] = (
        jnp.dot(xb, wc_ref[...], preferred_element_type=jnp.float32) + b1_ref[...]
    )
    n_ref[...] = jnp.dot(xb, wb_ref[...], preferred_element_type=jnp.float32)

    xsq = xb * xb
    ones_row = jnp.ones((1, xb.shape[1]), jnp.float32)
    sq_row = lax.dot_general(
        ones_row, xsq, (((1,), (1,)), ((), ())),
        preferred_element_type=jnp.float32)
    gram = lax.dot_general(
        xb, xb, (((1,), (1,)), ((), ())),
        preferred_element_type=jnp.float32)
    dmat = sq_row - 2.0 * gram

    col = lax.broadcasted_iota(jnp.int32, (_P, _P), 1)
    for t in range(_K):
        sel = jnp.argmin(dmat, axis=1).astype(jnp.int32)
        idx_ref[t, :] = sel
        if t + 1 < _K:
            dmat = jnp.where(col == sel[:, None], jnp.inf, dmat)


def _knn_proj(x, wc, wb, b1):
    d_in = x.shape[1]
    d_out = wc.shape[1]
    d_n = wb.shape[1]
    return pl.pallas_call(
        _knn_proj_body,
        out_shape=[
            jax.ShapeDtypeStruct((_P, d_out), jnp.float32),
            jax.ShapeDtypeStruct((_P, d_n), jnp.float32),
            jax.ShapeDtypeStruct((_K, _P), jnp.int32),
        ],
    )(x, wc, wb, b1)


@functools.lru_cache(maxsize=None)
def _make_sc_gather(d_out):
    info = plsc.get_sparse_core_info()
    nw = info.num_cores * info.num_subcores
    e_total = _P * _K
    per_w = e_total // nw
    chunk = {512: 64, 256: 160, 128: 320}[d_out]
    n_iter = per_w // chunk
    mesh = plsc.VectorSubcoreMesh(core_axis_name="c", subcore_axis_name="s")

    @functools.partial(
        pl.kernel,
        mesh=mesh,
        out_type=jax.ShapeDtypeStruct((e_total, d_out), jnp.float32),
        scratch_types=[
            pltpu.VMEM((chunk,), jnp.int32),
            pltpu.VMEM((chunk,), jnp.int32),
            pltpu.VMEM((chunk, d_out), jnp.float32),
            pltpu.VMEM((chunk, d_out), jnp.float32),
            pltpu.SemaphoreType.DMA,
            pltpu.SemaphoreType.DMA,
        ],
    )
    def gather(table_hbm, idx_hbm, out_hbm, idx0, idx1, rows0, rows1,
               sem0, sem1):
        wid = lax.axis_index("s") * info.num_cores + lax.axis_index("c")
        base = wid * per_w
        idxs = (idx0, idx1)
        rows = (rows0, rows1)
        sems = (sem0, sem1)

        for b in range(2):
            pltpu.sync_copy(idx_hbm.at[pl.ds(base + b * chunk, chunk)],
                            idxs[b])
            pltpu.async_copy(table_hbm.at[idxs[b]], rows[b], sems[b])

        def outer(g, carry):
            for b in range(2):
                j = g + b
                pltpu.make_async_copy(
                    table_hbm.at[idxs[b]], rows[b], sems[b]).wait()
                pltpu.sync_copy(rows[b],
                                out_hbm.at[pl.ds(base + j * chunk, chunk)])

                @pl.when(j + 2 < n_iter)
                def _():
                    pltpu.sync_copy(
                        idx_hbm.at[pl.ds(base + (j + 2) * chunk, chunk)],
                        idxs[b])
                    pltpu.async_copy(table_hbm.at[idxs[b]], rows[b],
                                     sems[b])
            return carry

        lax.fori_loop(0, n_iter // 2, lambda i, c: outer(i * 2, c), 0,
                      unroll=False)

    return gather


_PB = 128
_NPB = _P // _PB


def _edge_mlp_body(g_ref, c_ref, w2_ref, b2_ref, o_ref):
    d = c_ref.shape[1]
    g3 = g_ref[:, :, :d]
    h1 = jnp.maximum(g3 + c_ref[...][None, :, :], 0.0)
    h1m = h1.reshape(_K * _PB, d)
    h2 = jnp.maximum(
        jnp.dot(h1m, w2_ref[...], preferred_element_type=jnp.float32)
        + b2_ref[...], 0.0)
    o_ref[...] = jnp.max(h2.reshape(_K, _PB, d), axis=0)


def _edge_mlp(g3, c, w2, b2):
    d = c.shape[1]
    d_g = g3.shape[-1]
    return pl.pallas_call(
        _edge_mlp_body,
        grid=(_NPB,),
        in_specs=[
            pl.BlockSpec((_K, _PB, d_g), lambda pb: (0, pb, 0)),
            pl.BlockSpec((_PB, d), lambda pb: (pb, 0)),
            pl.BlockSpec((d, d), lambda pb: (0, 0)),
            pl.BlockSpec((1, d), lambda pb: (0, 0)),
        ],
        out_specs=pl.BlockSpec((_PB, d), lambda pb: (pb, 0)),
        out_shape=jax.ShapeDtypeStruct((_P, d), jnp.float32),
    )(g3, c, w2, b2)


_MB = 512


def _final_mlp_body(xc_ref, w1_ref, b1_ref, w2_ref, b2_ref, w3_ref, b3_ref, o_ref):
    h = jnp.maximum(
        jnp.dot(xc_ref[...], w1_ref[...], preferred_element_type=jnp.float32)
        + b1_ref[...], 0.0)
    h = jnp.maximum(
        jnp.dot(h, w2_ref[...], preferred_element_type=jnp.float32)
        + b2_ref[...], 0.0)
    o_ref[...] = jax.nn.sigmoid(
        jnp.dot(h, w3_ref[...], preferred_element_type=jnp.float32)
        + b3_ref[...])


def _final_mlp(xc, w1, b1, w2, b2, w3, b3):
    d1, d2, d3 = w1.shape[1], w2.shape[1], w3.shape[1]
    din = xc.shape[1]
    return pl.pallas_call(
        _final_mlp_body,
        grid=(xc.shape[0] // _MB,),
        in_specs=[
            pl.BlockSpec((_MB, din), lambda r: (r, 0)),
            pl.BlockSpec((din, d1), lambda r: (0, 0)),
            pl.BlockSpec((1, d1), lambda r: (0, 0)),
            pl.BlockSpec((d1, d2), lambda r: (0, 0)),
            pl.BlockSpec((1, d2), lambda r: (0, 0)),
            pl.BlockSpec((d2, d3), lambda r: (0, 0)),
            pl.BlockSpec((1, d3), lambda r: (0, 0)),
        ],
        out_specs=pl.BlockSpec((_MB, d3), lambda r: (r, 0)),
        out_shape=jax.ShapeDtypeStruct((xc.shape[0], d3), jnp.float32),
    )(xc, w1, b1, w2, b2, w3, b3)


def _prep_w(w1, d_in):
    wc = w1[:d_in] - w1[d_in:]
    wb = w1[d_in:]
    if w1.shape[1] < 128:
        wb = jnp.pad(wb, ((0, 0), (0, 128 - w1.shape[1])))
    return wc, wb


def _edge_conv_cloud(xc, wc, wb, b1r, w2, b2r):
    c, n, idx = _knn_proj(xc, wc, wb, b1r)
    g = _make_sc_gather(n.shape[1])(n, idx.reshape(-1))
    g3 = g.reshape(_K, _P, n.shape[1])
    return _edge_mlp(g3, c, w2, b2r)


def kernel(x, batch, W1a, b1a, W1b, b1b, W2a, b2a, W2b, b2b, W3a, b3a, W3b, b3b,
           W4a, b4a, W4b, b4b, Wf1, bf1, Wf2, bf2, Wf3, bf3):
    layer_ws = []
    for (w1, b1, w2, b2), d_in in zip(
            [(W1a, b1a, W1b, b1b), (W2a, b2a, W2b, b2b),
             (W3a, b3a, W3b, b3b), (W4a, b4a, W4b, b4b)],
            [3, 64, 128, 256]):
        wc, wb = _prep_w(w1, d_in)
        layer_ws.append((wc, wb, b1.reshape(1, -1), w2, b2.reshape(1, -1)))
    w3p = jnp.pad(Wf3, ((0, 0), (0, 127)))
    b3p = jnp.pad(bf3, (0, 127)).reshape(1, -1)
    bf1r, bf2r = bf1.reshape(1, -1), bf2.reshape(1, -1)

    outs = []
    for cl in range(_NB):
        h = lax.slice_in_dim(x, cl * _P, (cl + 1) * _P, axis=0)
        feats = []
        for wc, wb, b1r, w2, b2r in layer_ws:
            h = _edge_conv_cloud(h, wc, wb, b1r, w2, b2r)
            feats.append(h)
        xcat = jnp.concatenate(feats, axis=1)
        o = _final_mlp(xcat, Wf1, bf1r, Wf2, bf2r, w3p, b3p)
        outs.append(o[:, :1])
    return jnp.concatenate(outs, axis=0)


# speedup vs baseline: 1.0833x; 1.0833x over previous
import functools

import jax
import jax.numpy as jnp
from jax import lax
from jax.experimental import pallas as pl
from jax.experimental.pallas import tpu as pltpu
from jax.experimental.pallas import tpu_sc as plsc

_K = 20
_NB = 4
_P = 1024
_N = _NB * _P


def _knn_proj_body(x_ref, wc_ref, wb_ref, b1_ref, c_ref, n_ref, idx_ref):
    xb = x_ref[...]
    c_ref[...] = (
        jnp.dot(xb, wc_ref[...], preferred_element_type=jnp.float32) + b1_ref[...]
    )
    n_ref[...] = jnp.dot(xb, wb_ref[...], preferred_element_type=jnp.float32)

    xsq = xb * xb
    ones_row = jnp.ones((1, xb.shape[1]), jnp.float32)
    sq_row = lax.dot_general(
        ones_row, xsq, (((1,), (1,)), ((), ())),
        preferred_element_type=jnp.float32)
    gram = lax.dot_general(
        xb, xb, (((1,), (1,)), ((), ())),
        preferred_element_type=jnp.float32)
    dmat = sq_row - 2.0 * gram

    col = lax.broadcasted_iota(jnp.int32, (_P, _P), 1)
    for t in range(_K):
        sel = jnp.argmin(dmat, axis=1).astype(jnp.int32)
        idx_ref[t, :] = sel
        if t + 1 < _K:
            dmat = jnp.where(col == sel[:, None], jnp.inf, dmat)


def _knn_proj(x, wc, wb, b1):
    d_in = x.shape[1]
    d_out = wc.shape[1]
    d_n = wb.shape[1]
    return pl.pallas_call(
        _knn_proj_body,
        out_shape=[
            jax.ShapeDtypeStruct((_P, d_out), jnp.float32),
            jax.ShapeDtypeStruct((_P, d_n), jnp.float32),
            jax.ShapeDtypeStruct((_K, _P), jnp.int32),
        ],
    )(x, wc, wb, b1)


@functools.lru_cache(maxsize=None)
def _make_sc_gather(d_out):
    info = plsc.get_sparse_core_info()
    nw = info.num_cores * info.num_subcores
    e_total = _P * _K
    per_w = e_total // nw
    chunk = 64 if d_out >= 512 else 128
    n_iter = per_w // chunk
    mesh = plsc.VectorSubcoreMesh(core_axis_name="c", subcore_axis_name="s")

    @functools.partial(
        pl.kernel,
        mesh=mesh,
        out_type=jax.ShapeDtypeStruct((e_total, d_out), jnp.float32),
        scratch_types=[
            pltpu.VMEM((chunk,), jnp.int32),
            pltpu.VMEM((chunk, d_out), jnp.float32),
            pltpu.SemaphoreType.DMA,
        ],
    )
    def gather(table_hbm, idx_hbm, out_hbm, idx_v, rows_v, sem):
        wid = lax.axis_index("s") * info.num_cores + lax.axis_index("c")
        base = wid * per_w

        def body(i, carry):
            off = base + i * chunk
            pltpu.sync_copy(idx_hbm.at[pl.ds(off, chunk)], idx_v)
            pltpu.async_copy(table_hbm.at[idx_v], rows_v, sem).wait()
            pltpu.sync_copy(rows_v, out_hbm.at[pl.ds(off, chunk)])
            return carry

        lax.fori_loop(0, n_iter, body, 0)

    return gather


_PB = 128
_NPB = _P // _PB


def _edge_mlp_body(g_ref, c_ref, w2_ref, b2_ref, o_ref):
    d = c_ref.shape[1]
    g3 = g_ref[:, :, :d]
    h1 = jnp.maximum(g3 + c_ref[...][None, :, :], 0.0)
    h1m = h1.reshape(_K * _PB, d)
    h2 = jnp.maximum(
        jnp.dot(h1m, w2_ref[...], preferred_element_type=jnp.float32)
        + b2_ref[...], 0.0)
    o_ref[...] = jnp.max(h2.reshape(_K, _PB, d), axis=0)


def _edge_mlp(g3, c, w2, b2):
    d = c.shape[1]
    d_g = g3.shape[-1]
    return pl.pallas_call(
        _edge_mlp_body,
        grid=(_NPB,),
        in_specs=[
            pl.BlockSpec((_K, _PB, d_g), lambda pb: (0, pb, 0)),
            pl.BlockSpec((_PB, d), lambda pb: (pb, 0)),
            pl.BlockSpec((d, d), lambda pb: (0, 0)),
            pl.BlockSpec((1, d), lambda pb: (0, 0)),
        ],
        out_specs=pl.BlockSpec((_PB, d), lambda pb: (pb, 0)),
        out_shape=jax.ShapeDtypeStruct((_P, d), jnp.float32),
    )(g3, c, w2, b2)


_MB = 512


def _final_mlp_body(x1_ref, x2_ref, x3_ref, x4_ref,
                    w1a_ref, w1b_ref, w1c_ref, w1d_ref, b1_ref,
                    w2_ref, b2_ref, w3_ref, b3_ref, o_ref):
    acc = jnp.dot(x1_ref[...], w1a_ref[...], preferred_element_type=jnp.float32)
    acc = acc + jnp.dot(x2_ref[...], w1b_ref[...],
                        preferred_element_type=jnp.float32)
    acc = acc + jnp.dot(x3_ref[...], w1c_ref[...],
                        preferred_element_type=jnp.float32)
    acc = acc + jnp.dot(x4_ref[...], w1d_ref[...],
                        preferred_element_type=jnp.float32)
    h = jnp.maximum(acc + b1_ref[...], 0.0)
    h = jnp.maximum(
        jnp.dot(h, w2_ref[...], preferred_element_type=jnp.float32)
        + b2_ref[...], 0.0)
    o_ref[...] = jax.nn.sigmoid(
        jnp.dot(h, w3_ref[...], preferred_element_type=jnp.float32)
        + b3_ref[...])


def _final_mlp(x1, x2, x3, x4, w1s, b1, w2, b2, w3, b3):
    d1, d2, d3 = w1s[0].shape[1], w2.shape[1], w3.shape[1]
    xs = [x1, x2, x3, x4]
    in_specs = [pl.BlockSpec((_MB, xi.shape[1]), lambda r: (r, 0)) for xi in xs]
    in_specs += [pl.BlockSpec(w.shape, lambda r: (0, 0)) for w in w1s]
    in_specs += [
        pl.BlockSpec((1, d1), lambda r: (0, 0)),
        pl.BlockSpec((d1, d2), lambda r: (0, 0)),
        pl.BlockSpec((1, d2), lambda r: (0, 0)),
        pl.BlockSpec((d2, d3), lambda r: (0, 0)),
        pl.BlockSpec((1, d3), lambda r: (0, 0)),
    ]
    return pl.pallas_call(
        _final_mlp_body,
        grid=(_P // _MB,),
        in_specs=in_specs,
        out_specs=pl.BlockSpec((_MB, d3), lambda r: (r, 0)),
        out_shape=jax.ShapeDtypeStruct((_P, d3), jnp.float32),
    )(*xs, *w1s, b1, w2, b2, w3, b3)


def _prep_w(w1, d_in):
    wc = w1[:d_in] - w1[d_in:]
    wb = w1[d_in:]
    if w1.shape[1] < 128:
        wb = jnp.pad(wb, ((0, 0), (0, 128 - w1.shape[1])))
    return wc, wb


def _edge_conv_cloud(xc, wc, wb, b1r, w2, b2r):
    c, n, idx = _knn_proj(xc, wc, wb, b1r)
    g = _make_sc_gather(n.shape[1])(n, idx.reshape(-1))
    g3 = g.reshape(_K, _P, n.shape[1])
    return _edge_mlp(g3, c, w2, b2r)


def kernel(x, batch, W1a, b1a, W1b, b1b, W2a, b2a, W2b, b2b, W3a, b3a, W3b, b3b,
           W4a, b4a, W4b, b4b, Wf1, bf1, Wf2, bf2, Wf3, bf3):
    layer_ws = []
    for (w1, b1, w2, b2), d_in in zip(
            [(W1a, b1a, W1b, b1b), (W2a, b2a, W2b, b2b),
             (W3a, b3a, W3b, b3b), (W4a, b4a, W4b, b4b)],
            [3, 64, 128, 256]):
        wc, wb = _prep_w(w1, d_in)
        layer_ws.append((wc, wb, b1.reshape(1, -1), w2, b2.reshape(1, -1)))
    w3p = jnp.pad(Wf3, ((0, 0), (0, 127)))
    b3p = jnp.pad(bf3, (0, 127)).reshape(1, -1)
    bf1r, bf2r = bf1.reshape(1, -1), bf2.reshape(1, -1)
    w1s = (Wf1[:64], Wf1[64:192], Wf1[192:448], Wf1[448:960])

    outs = []
    for cl in range(_NB):
        h = lax.slice_in_dim(x, cl * _P, (cl + 1) * _P, axis=0)
        feats = []
        for wc, wb, b1r, w2, b2r in layer_ws:
            h = _edge_conv_cloud(h, wc, wb, b1r, w2, b2r)
            feats.append(h)
        o = _final_mlp(*feats, w1s, bf1r, Wf2, bf2r, w3p, b3p)
        outs.append(o[:, :1])
    return jnp.concatenate(outs, axis=0)


# speedup vs baseline: 1.1249x; 1.0384x over previous
import functools

import jax
import jax.numpy as jnp
from jax import lax
from jax.experimental import pallas as pl
from jax.experimental.pallas import tpu as pltpu
from jax.experimental.pallas import tpu_sc as plsc

_K = 20
_NB = 4
_P = 1024
_N = _NB * _P


def _knn_proj_body(x_ref, wc_ref, wb_ref, b1_ref, c_ref, n_ref, idx_ref):
    xb = x_ref[...]
    c_ref[...] = (
        jnp.dot(xb, wc_ref[...], preferred_element_type=jnp.float32) + b1_ref[...]
    )
    n_ref[...] = jnp.dot(xb, wb_ref[...], preferred_element_type=jnp.float32)

    xsq = xb * xb
    ones_row = jnp.ones((1, xb.shape[1]), jnp.float32)
    sq_row = lax.dot_general(
        ones_row, xsq, (((1,), (1,)), ((), ())),
        preferred_element_type=jnp.float32)
    gram = lax.dot_general(
        xb, xb, (((1,), (1,)), ((), ())),
        preferred_element_type=jnp.float32)
    dmat = sq_row - 2.0 * gram

    col = lax.broadcasted_iota(jnp.int32, (_P, _P), 1)
    for t in range(_K):
        sel = jnp.argmin(dmat, axis=1).astype(jnp.int32)
        idx_ref[t, :] = sel
        if t + 1 < _K:
            dmat = jnp.where(col == sel[:, None], jnp.inf, dmat)


def _knn_proj(x, wc, wb, b1):
    d_in = x.shape[1]
    d_out = wc.shape[1]
    d_n = wb.shape[1]
    return pl.pallas_call(
        _knn_proj_body,
        out_shape=[
            jax.ShapeDtypeStruct((_P, d_out), jnp.float32),
            jax.ShapeDtypeStruct((_P, d_n), jnp.float32),
            jax.ShapeDtypeStruct((_K, _P), jnp.int32),
        ],
    )(x, wc, wb, b1)


@functools.lru_cache(maxsize=None)
def _make_sc_gather(d_out):
    info = plsc.get_sparse_core_info()
    nw = info.num_cores * info.num_subcores
    e_total = _P * _K
    per_w = e_total // nw
    chunk = 64 if d_out >= 512 else 128
    n_iter = per_w // chunk
    mesh = plsc.VectorSubcoreMesh(core_axis_name="c", subcore_axis_name="s")

    @functools.partial(
        pl.kernel,
        mesh=mesh,
        out_type=jax.ShapeDtypeStruct((e_total, d_out), jnp.float32),
        scratch_types=[
            pltpu.VMEM((chunk,), jnp.int32),
            pltpu.VMEM((chunk, d_out), jnp.float32),
            pltpu.SemaphoreType.DMA,
        ],
    )
    def gather(table_hbm, idx_hbm, out_hbm, idx_v, rows_v, sem):
        wid = lax.axis_index("s") * info.num_cores + lax.axis_index("c")
        base = wid * per_w

        def body(i, carry):
            off = base + i * chunk
            pltpu.sync_copy(idx_hbm.at[pl.ds(off, chunk)], idx_v)
            pltpu.async_copy(table_hbm.at[idx_v], rows_v, sem).wait()
            pltpu.sync_copy(rows_v, out_hbm.at[pl.ds(off, chunk)])
            return carry

        lax.fori_loop(0, n_iter, body, 0)

    return gather


_PB = 256
_NPB = _P // _PB


def _edge_mlp_body(g_ref, c_ref, w2_ref, b2_ref, o_ref):
    d = c_ref.shape[1]
    g3 = g_ref[:, :, :d]
    h1 = jnp.maximum(g3 + c_ref[...][None, :, :], 0.0)
    h1m = h1.reshape(_K * _PB, d)
    h2 = jnp.maximum(
        jnp.dot(h1m, w2_ref[...], preferred_element_type=jnp.float32)
        + b2_ref[...], 0.0)
    o_ref[...] = jnp.max(h2.reshape(_K, _PB, d), axis=0)


def _edge_mlp(g3, c, w2, b2):
    d = c.shape[1]
    d_g = g3.shape[-1]
    return pl.pallas_call(
        _edge_mlp_body,
        grid=(_NPB,),
        in_specs=[
            pl.BlockSpec((_K, _PB, d_g), lambda pb: (0, pb, 0)),
            pl.BlockSpec((_PB, d), lambda pb: (pb, 0)),
            pl.BlockSpec((d, d), lambda pb: (0, 0)),
            pl.BlockSpec((1, d), lambda pb: (0, 0)),
        ],
        out_specs=pl.BlockSpec((_PB, d), lambda pb: (pb, 0)),
        out_shape=jax.ShapeDtypeStruct((_P, d), jnp.float32),
    )(g3, c, w2, b2)


_MB = 512


def _final_mlp_body(x1_ref, x2_ref, x3_ref, x4_ref,
                    w1a_ref, w1b_ref, w1c_ref, w1d_ref, b1_ref,
                    w2_ref, b2_ref, w3_ref, b3_ref, o_ref):
    acc = jnp.dot(x1_ref[...], w1a_ref[...], preferred_element_type=jnp.float32)
    acc = acc + jnp.dot(x2_ref[...], w1b_ref[...],
                        preferred_element_type=jnp.float32)
    acc = acc + jnp.dot(x3_ref[...], w1c_ref[...],
                        preferred_element_type=jnp.float32)
    acc = acc + jnp.dot(x4_ref[...], w1d_ref[...],
                        preferred_element_type=jnp.float32)
    h = jnp.maximum(acc + b1_ref[...], 0.0)
    h = jnp.maximum(
        jnp.dot(h, w2_ref[...], preferred_element_type=jnp.float32)
        + b2_ref[...], 0.0)
    o_ref[...] = jax.nn.sigmoid(
        jnp.dot(h, w3_ref[...], preferred_element_type=jnp.float32)
        + b3_ref[...])


def _final_mlp(x1, x2, x3, x4, w1s, b1, w2, b2, w3, b3):
    d1, d2, d3 = w1s[0].shape[1], w2.shape[1], w3.shape[1]
    xs = [x1, x2, x3, x4]
    in_specs = [pl.BlockSpec((_MB, xi.shape[1]), lambda r: (r, 0)) for xi in xs]
    in_specs += [pl.BlockSpec(w.shape, lambda r: (0, 0)) for w in w1s]
    in_specs += [
        pl.BlockSpec((1, d1), lambda r: (0, 0)),
        pl.BlockSpec((d1, d2), lambda r: (0, 0)),
        pl.BlockSpec((1, d2), lambda r: (0, 0)),
        pl.BlockSpec((d2, d3), lambda r: (0, 0)),
        pl.BlockSpec((1, d3), lambda r: (0, 0)),
    ]
    return pl.pallas_call(
        _final_mlp_body,
        grid=(_P // _MB,),
        in_specs=in_specs,
        out_specs=pl.BlockSpec((_MB, d3), lambda r: (r, 0)),
        out_shape=jax.ShapeDtypeStruct((_P, d3), jnp.float32),
    )(*xs, *w1s, b1, w2, b2, w3, b3)


def _prep_w(w1, d_in):
    wc = w1[:d_in] - w1[d_in:]
    wb = w1[d_in:]
    if w1.shape[1] < 128:
        wb = jnp.pad(wb, ((0, 0), (0, 128 - w1.shape[1])))
    return wc, wb


def _edge_conv_cloud(xc, wc, wb, b1r, w2, b2r):
    c, n, idx = _knn_proj(xc, wc, wb, b1r)
    g = _make_sc_gather(n.shape[1])(n, idx.reshape(-1))
    g3 = g.reshape(_K, _P, n.shape[1])
    return _edge_mlp(g3, c, w2, b2r)


def kernel(x, batch, W1a, b1a, W1b, b1b, W2a, b2a, W2b, b2b, W3a, b3a, W3b, b3b,
           W4a, b4a, W4b, b4b, Wf1, bf1, Wf2, bf2, Wf3, bf3):
    layer_ws = []
    for (w1, b1, w2, b2), d_in in zip(
            [(W1a, b1a, W1b, b1b), (W2a, b2a, W2b, b2b),
             (W3a, b3a, W3b, b3b), (W4a, b4a, W4b, b4b)],
            [3, 64, 128, 256]):
        wc, wb = _prep_w(w1, d_in)
        layer_ws.append((wc, wb, b1.reshape(1, -1), w2, b2.reshape(1, -1)))
    w3p = jnp.pad(Wf3, ((0, 0), (0, 127)))
    b3p = jnp.pad(bf3, (0, 127)).reshape(1, -1)
    bf1r, bf2r = bf1.reshape(1, -1), bf2.reshape(1, -1)
    w1s = (Wf1[:64], Wf1[64:192], Wf1[192:448], Wf1[448:960])

    outs = []
    for cl in range(_NB):
        h = lax.slice_in_dim(x, cl * _P, (cl + 1) * _P, axis=0)
        feats = []
        for wc, wb, b1r, w2, b2r in layer_ws:
            h = _edge_conv_cloud(h, wc, wb, b1r, w2, b2r)
            feats.append(h)
        o = _final_mlp(*feats, w1s, bf1r, Wf2, bf2r, w3p, b3p)
        outs.append(o[:, :1])
    return jnp.concatenate(outs, axis=0)
